# baseline (speedup 1.0000x reference)
import jax
import jax.numpy as jnp
from jax import lax
from jax.experimental import pallas as pl
from jax.experimental.pallas import tpu as pltpu
from jax.experimental.pallas import tpu_sc as plsc

NC = 2
NS = 16
NW = NC * NS
L = 16

D = 128
S = 4096
T = 50
S_PER_W = S // NW
CH_S = 8
NCHUNK = S_PER_W // CH_S

_GROUP_OFF = (0, 16, 32, T - L)


def _worker_body(table, x, zrow, out, idx_v, buf0, buf1, g0, g1, s0, s1):
    wid = lax.axis_index("s") * NC + lax.axis_index("c")
    sent0 = wid * S_PER_W
    bufs = (buf0, buf1)
    gsems = (g0, g1)
    ssems = (s0, s1)

    pltpu.sync_copy(x.at[pl.ds(sent0, S_PER_W)], idx_v)

    def gather_parts(c, k):
        for j in range(CH_S):
            yield (table.at[idx_v.at[c * CH_S + j]], bufs[k].at[j], gsems[k])

    def start_gather(c, k):
        for src, dst, sem in gather_parts(c, k):
            pltpu.async_copy(src, dst, sem)

    def wait_gather(c, k):
        for src, dst, sem in gather_parts(c, k):
            pltpu.make_async_copy(src, dst, sem).wait()

    def out_slice(c):
        return out.at[pl.ds(sent0 + c * CH_S, CH_S)]

    def start_scatter(c, k):
        pltpu.async_copy(bufs[k], out_slice(c), ssems[k])

    def wait_scatter(c, k):
        pltpu.make_async_copy(bufs[k], out_slice(c), ssems[k]).wait()

    def process(c, k):
        wait_gather(c, k)
        start_scatter(c, k)

    start_gather(0, 0)
    process(0, 0)
    start_gather(1, 1)

    @pl.loop(0, (NCHUNK - 2) // 2)
    def _steady(i):
        c1 = 2 * i + 1
        process(c1, 1)
        wait_scatter(c1 - 1, 0)
        start_gather(c1 + 1, 0)
        c2 = 2 * i + 2
        process(c2, 0)
        wait_scatter(c2 - 1, 1)
        start_gather(c2 + 1, 1)

    process(NCHUNK - 1, 1)
    wait_scatter(NCHUNK - 2, 0)
    wait_scatter(NCHUNK - 1, 1)

    @pl.loop(0, S_PER_W)
    def _fix_sent(sl):
        vs = [idx_v[sl, pl.ds(off, L)] for off in _GROUP_OFF]
        smn = vs[0]
        for v in vs[1:]:
            smn = jnp.minimum(smn, v)
        sent_pad = plsc.all_reduce_population_count(smn == 0)[0] > 0

        @pl.when(sent_pad)
        def _patch():
            for g, off in enumerate(_GROUP_OFF):
                for lane in range(L):
                    row = off + lane

                    @pl.when(vs[g][lane] == 0)
                    def _zero_row():
                        pltpu.sync_copy(zrow.at[0], out.at[sent0 + sl, row])


@jax.jit
def kernel(embedding, x):
    xi = x.astype(jnp.int32)
    zrow = jnp.zeros((8, D), jnp.float32)
    mesh = plsc.VectorSubcoreMesh(
        core_axis_name="c", subcore_axis_name="s",
        num_cores=NC, num_subcores=NS,
    )
    return pl.kernel(
        _worker_body,
        out_type=jax.ShapeDtypeStruct((S, T, D), jnp.float32),
        mesh=mesh,
        compiler_params=pltpu.CompilerParams(needs_layout_passes=False),
        scratch_types=[
            pltpu.VMEM((S_PER_W, T), jnp.int32),
            pltpu.VMEM((CH_S, T, D), jnp.float32),
            pltpu.VMEM((CH_S, T, D), jnp.float32),
            pltpu.SemaphoreType.DMA,
            pltpu.SemaphoreType.DMA,
            pltpu.SemaphoreType.DMA,
            pltpu.SemaphoreType.DMA,
        ],
    )(embedding, xi, zrow)
